# baseline (speedup 1.0000x reference)
import functools

import jax
import jax.numpy as jnp
from jax import lax
from jax.experimental import pallas as pl
from jax.experimental.pallas import tpu as pltpu, tpu_sc as plsc

N_NODES = 10000
N_EDGES = 320000
D_FEAT = 128
CHUNK = 1000
NC, NS = 2, 16
NW = NC * NS
E_PER = N_EDGES // NW
LANES = 16

_sc_mesh = plsc.VectorSubcoreMesh(core_axis_name="c", subcore_axis_name="s")


def _matvec_body(x_ref, w2_ref, b_ref, y_ref, base_ref):
    xv = x_ref[...]
    wrel = w2_ref[0:1, :]
    wroot = w2_ref[1:2, :]
    y_ref[...] = jnp.sum(xv * wrel, axis=1)
    base_ref[...] = jnp.sum(xv * wroot, axis=1) + b_ref[0, 0] - xv[:, 0]


def _reduce_body(p_ref, base_ref, o_ref):
    o_ref[...] = jnp.sum(p_ref[...], axis=0) + base_ref[...]


@functools.partial(
    pl.kernel,
    out_type=jax.ShapeDtypeStruct((NW, N_NODES), jnp.float32),
    mesh=_sc_mesh,
    scratch_types=[
        pltpu.VMEM((N_NODES,), jnp.float32),
        pltpu.VMEM((N_NODES,), jnp.float32),
        pltpu.VMEM((E_PER,), jnp.int32),
        pltpu.VMEM((E_PER,), jnp.int32),
        pltpu.VMEM((E_PER,), jnp.float32),
        pltpu.SemaphoreType.DMA,
    ],
    compiler_params=pltpu.CompilerParams(needs_layout_passes=False),
)
def _sc_edge_scatter(y_hbm, ei_hbm, ea_hbm, out_hbm,
                     y_v, acc_v, src_v, dst_v, ea_v, sem):
    cid = lax.axis_index("c")
    sid = lax.axis_index("s")
    wid = sid * NC + cid
    e0 = wid * E_PER

    c_y = pltpu.async_copy(y_hbm, y_v, sem)
    c_s = pltpu.async_copy(ei_hbm.at[pl.ds(e0, E_PER)], src_v, sem)
    c_d = pltpu.async_copy(ei_hbm.at[pl.ds(N_EDGES + e0, E_PER)], dst_v, sem)
    c_e = pltpu.async_copy(ea_hbm.at[pl.ds(e0, E_PER)], ea_v, sem)

    @plsc.parallel_loop(0, N_NODES // LANES, unroll=8)
    def _zero(i):
        acc_v[pl.ds(i * LANES, LANES)] = jnp.zeros((LANES,), jnp.float32)

    c_y.wait()
    c_s.wait()
    c_d.wait()
    c_e.wait()

    @plsc.parallel_loop(0, E_PER // LANES, unroll=16)
    def _edges(i):
        off = i * LANES
        sv = src_v[pl.ds(off, LANES)]
        dv = dst_v[pl.ds(off, LANES)]
        ev = ea_v[pl.ds(off, LANES)]
        yv = plsc.load_gather(y_v, [sv])
        plsc.addupdate_scatter(acc_v, [dv], ev * yv)

    pltpu.sync_copy(acc_v, out_hbm.at[wid])


def kernel(x, edge_index, edge_attr, batch, W_rel, b_rel, W_root):
    ei = edge_index.astype(jnp.int32).reshape(2 * N_EDGES)
    w2 = jnp.stack([W_rel[0], W_root[0]])
    b0 = b_rel[0].reshape(1, 1)

    y, base = pl.pallas_call(
        _matvec_body,
        out_shape=[
            jax.ShapeDtypeStruct((N_NODES,), jnp.float32),
            jax.ShapeDtypeStruct((N_NODES,), jnp.float32),
        ],
        in_specs=[
            pl.BlockSpec(memory_space=pltpu.VMEM),
            pl.BlockSpec(memory_space=pltpu.VMEM),
            pl.BlockSpec(memory_space=pltpu.SMEM),
        ],
        out_specs=[
            pl.BlockSpec(memory_space=pltpu.VMEM),
            pl.BlockSpec(memory_space=pltpu.VMEM),
        ],
    )(x, w2, b0)

    partials = _sc_edge_scatter(y, ei, edge_attr)

    out = pl.pallas_call(
        _reduce_body,
        out_shape=jax.ShapeDtypeStruct((N_NODES,), jnp.float32),
    )(partials, base)

    return out.reshape(N_NODES // CHUNK, CHUNK)


# speedup vs baseline: 1.0703x; 1.0703x over previous
import functools

import jax
import jax.numpy as jnp
from jax import lax
from jax.experimental import pallas as pl
from jax.experimental.pallas import tpu as pltpu, tpu_sc as plsc

N_NODES = 10000
N_EDGES = 320000
D_FEAT = 128
CHUNK = 1000
NC, NS = 2, 16
NW = NC * NS
E_PER = N_EDGES // NW
LANES = 16
ROWS_PER = 624
ROWS_MAX = 640
W_LEN = 2 * D_FEAT + LANES

_sc_mesh = plsc.VectorSubcoreMesh(core_axis_name="c", subcore_axis_name="s")


def _reduce_body(p_ref, o_ref):
    o_ref[...] = jnp.sum(p_ref[...], axis=0)


@functools.partial(
    pl.kernel,
    out_type=jax.ShapeDtypeStruct((NW, N_NODES), jnp.float32),
    mesh=_sc_mesh,
    scratch_types=[
        pltpu.VMEM((N_NODES,), jnp.float32),
        pltpu.VMEM((N_NODES,), jnp.float32),
        pltpu.VMEM((ROWS_MAX,), jnp.float32),
        pltpu.VMEM((ROWS_MAX,), jnp.float32),
        pltpu.VMEM((W_LEN,), jnp.float32),
        pltpu.VMEM_SHARED((N_NODES,), jnp.float32),
        pltpu.SemaphoreType.DMA,
    ],
    compiler_params=pltpu.CompilerParams(needs_layout_passes=False),
)
def _sc_fused(x_hbm, w_hbm, ei_hbm, ea_hbm, out_hbm,
              y_full, acc_v, y_loc, base_loc, w_v, y_sh, sem):
    cid = lax.axis_index("c")
    sid = lax.axis_index("s")
    wid = sid * NC + cid
    row0 = sid * ROWS_PER

    pltpu.sync_copy(w_hbm, w_v)

    def phase_a(x_v):
        c_x = pltpu.async_copy(
            x_hbm.at[pl.ds(row0 * D_FEAT, ROWS_MAX * D_FEAT)], x_v, sem)

        @plsc.parallel_loop(0, N_NODES // LANES, unroll=8)
        def _zero(i):
            acc_v[pl.ds(i * LANES, LANES)] = jnp.zeros((LANES,), jnp.float32)

        c_x.wait()

        wy = [w_v[pl.ds(f * LANES, LANES)] for f in range(D_FEAT // LANES)]
        wz = [w_v[pl.ds(D_FEAT + f * LANES, LANES)]
              for f in range(D_FEAT // LANES)]
        b0 = w_v[pl.ds(2 * D_FEAT, LANES)][0]
        iot = lax.broadcasted_iota(jnp.int32, (LANES,), 0)
        zv = jnp.zeros((LANES,), jnp.float32)

        def _row_group(g, carry):
            yvec = zv
            bvec = zv
            for i in range(LANES):
                off = (g * LANES + i) * D_FEAT
                xc0 = x_v[pl.ds(off, LANES)]
                ay = xc0 * wy[0]
                az = xc0 * wz[0]
                for f in range(1, D_FEAT // LANES):
                    xc = x_v[pl.ds(off + f * LANES, LANES)]
                    ay = ay + xc * wy[f]
                    az = az + xc * wz[f]
                m = iot == i
                yvec = jnp.where(m, jnp.sum(ay), yvec)
                bvec = jnp.where(m, jnp.sum(az) + b0 - xc0[0], bvec)
            y_loc[pl.ds(g * LANES, LANES)] = yvec
            base_loc[pl.ds(g * LANES, LANES)] = bvec
            return carry

        lax.fori_loop(0, ROWS_MAX // LANES, _row_group, 0)

    pl.run_scoped(phase_a, pltpu.VMEM((ROWS_MAX * D_FEAT,), jnp.float32))

    pltpu.sync_copy(y_loc, y_sh.at[pl.ds(row0, ROWS_MAX)])
    plsc.subcore_barrier()
    pltpu.sync_copy(y_sh, y_full)

    nrows = ROWS_PER + jnp.where(sid == NS - 1, LANES, 0)

    @pl.when(cid == 0)
    def _add_base():
        def bb(j, carry):
            s = row0 + j * LANES
            acc_v[pl.ds(s, LANES)] = (
                acc_v[pl.ds(s, LANES)] + base_loc[pl.ds(j * LANES, LANES)])
            return carry

        lax.fori_loop(0, nrows // LANES, bb, 0)

    def phase_b(src_v, dst_v, ea_v):
        e0 = wid * E_PER
        c_s = pltpu.async_copy(ei_hbm.at[pl.ds(e0, E_PER)], src_v, sem)
        c_d = pltpu.async_copy(
            ei_hbm.at[pl.ds(N_EDGES + e0, E_PER)], dst_v, sem)
        c_e = pltpu.async_copy(ea_hbm.at[pl.ds(e0, E_PER)], ea_v, sem)
        c_s.wait()
        c_d.wait()
        c_e.wait()

        @plsc.parallel_loop(0, E_PER // LANES, unroll=8)
        def _edges(i):
            off = i * LANES
            sv = src_v[pl.ds(off, LANES)]
            dv = dst_v[pl.ds(off, LANES)]
            ev = ea_v[pl.ds(off, LANES)]
            yv = plsc.load_gather(y_full, [sv])
            plsc.addupdate_scatter(acc_v, [dv], ev * yv)

    pl.run_scoped(
        phase_b,
        pltpu.VMEM((E_PER,), jnp.int32),
        pltpu.VMEM((E_PER,), jnp.int32),
        pltpu.VMEM((E_PER,), jnp.float32),
    )

    pltpu.sync_copy(acc_v, out_hbm.at[wid])


def kernel(x, edge_index, edge_attr, batch, W_rel, b_rel, W_root):
    xf = x.reshape(N_NODES * D_FEAT)
    ei = edge_index.astype(jnp.int32).reshape(2 * N_EDGES)
    wflat = jnp.concatenate(
        [W_rel[0], W_root[0], jnp.broadcast_to(b_rel[0], (LANES,))])

    partials = _sc_fused(xf, wflat, ei, edge_attr)

    out = pl.pallas_call(
        _reduce_body,
        out_shape=jax.ShapeDtypeStruct((N_NODES,), jnp.float32),
    )(partials)

    return out.reshape(N_NODES // CHUNK, CHUNK)


# speedup vs baseline: 1.1367x; 1.0620x over previous
import functools

import jax
import jax.numpy as jnp
from jax import lax
from jax.experimental import pallas as pl
from jax.experimental.pallas import tpu as pltpu, tpu_sc as plsc

N_NODES = 10000
N_EDGES = 320000
D_FEAT = 128
CHUNK = 1000
NC, NS = 2, 16
NW = NC * NS
E_PER = N_EDGES // NW
LANES = 16
ROWS_PER = 624
ROWS_MAX = 640
XCHUNK = 160
NXCHUNK = ROWS_MAX // XCHUNK
W_LEN = 2 * D_FEAT + LANES

_sc_mesh = plsc.VectorSubcoreMesh(core_axis_name="c", subcore_axis_name="s")


def _reduce_body(p_ref, o_ref):
    o_ref[...] = jnp.sum(p_ref[...], axis=0)


@functools.partial(
    pl.kernel,
    out_type=jax.ShapeDtypeStruct((NW, N_NODES), jnp.float32),
    mesh=_sc_mesh,
    scratch_types=[
        pltpu.VMEM((N_NODES,), jnp.float32),
        pltpu.VMEM((N_NODES,), jnp.float32),
        pltpu.VMEM((ROWS_MAX,), jnp.float32),
        pltpu.VMEM((ROWS_MAX,), jnp.float32),
        pltpu.VMEM((W_LEN,), jnp.float32),
        pltpu.VMEM((XCHUNK * D_FEAT,), jnp.float32),
        pltpu.VMEM((XCHUNK * D_FEAT,), jnp.float32),
        pltpu.VMEM((E_PER,), jnp.int32),
        pltpu.VMEM((E_PER,), jnp.int32),
        pltpu.VMEM((E_PER,), jnp.float32),
        pltpu.VMEM_SHARED((N_NODES,), jnp.float32),
        pltpu.SemaphoreType.DMA,
        pltpu.SemaphoreType.DMA,
    ],
    compiler_params=pltpu.CompilerParams(needs_layout_passes=False),
)
def _sc_fused(x_hbm, w_hbm, ei_hbm, ea_hbm, out_hbm,
              y_full, acc_v, y_loc, base_loc, w_v, x_a, x_b,
              src_v, dst_v, ea_v, y_sh, sem_x, sem_e):
    cid = lax.axis_index("c")
    sid = lax.axis_index("s")
    wid = sid * NC + cid
    row0 = sid * ROWS_PER
    e0 = wid * E_PER

    c_s = pltpu.async_copy(ei_hbm.at[pl.ds(e0, E_PER)], src_v, sem_e)
    c_d = pltpu.async_copy(ei_hbm.at[pl.ds(N_EDGES + e0, E_PER)], dst_v, sem_e)
    c_e = pltpu.async_copy(ea_hbm.at[pl.ds(e0, E_PER)], ea_v, sem_e)

    pltpu.sync_copy(w_hbm, w_v)

    xbufs = [x_a, x_b]
    copies = [None, None]
    copies[0] = pltpu.async_copy(
        x_hbm.at[pl.ds(row0 * D_FEAT, XCHUNK * D_FEAT)], x_a, sem_x)

    @plsc.parallel_loop(0, N_NODES // LANES, unroll=8)
    def _zero(i):
        acc_v[pl.ds(i * LANES, LANES)] = jnp.zeros((LANES,), jnp.float32)

    wy = [w_v[pl.ds(f * LANES, LANES)] for f in range(D_FEAT // LANES)]
    wz = [w_v[pl.ds(D_FEAT + f * LANES, LANES)]
          for f in range(D_FEAT // LANES)]
    b0 = w_v[pl.ds(2 * D_FEAT, LANES)][0]
    iot = lax.broadcasted_iota(jnp.int32, (LANES,), 0)
    zv = jnp.zeros((LANES,), jnp.float32)

    for ck in range(NXCHUNK):
        copies[ck % 2].wait()
        if ck + 1 < NXCHUNK:
            copies[(ck + 1) % 2] = pltpu.async_copy(
                x_hbm.at[pl.ds((row0 + (ck + 1) * XCHUNK) * D_FEAT,
                               XCHUNK * D_FEAT)],
                xbufs[(ck + 1) % 2], sem_x)
        x_v = xbufs[ck % 2]
        loc0 = ck * XCHUNK

        def _row_group(g, carry, x_v=x_v, loc0=loc0):
            yvec = zv
            bvec = zv
            for i in range(LANES):
                off = (g * LANES + i) * D_FEAT
                xc0 = x_v[pl.ds(off, LANES)]
                ay = xc0 * wy[0]
                az = xc0 * wz[0]
                for f in range(1, D_FEAT // LANES):
                    xc = x_v[pl.ds(off + f * LANES, LANES)]
                    ay = ay + xc * wy[f]
                    az = az + xc * wz[f]
                m = iot == i
                yvec = jnp.where(m, jnp.sum(ay), yvec)
                bvec = jnp.where(m, jnp.sum(az) + b0 - xc0[0], bvec)
            y_loc[pl.ds(loc0 + g * LANES, LANES)] = yvec
            base_loc[pl.ds(loc0 + g * LANES, LANES)] = bvec
            return carry

        lax.fori_loop(0, XCHUNK // LANES, _row_group, 0)

    pltpu.sync_copy(y_loc, y_sh.at[pl.ds(row0, ROWS_MAX)])
    plsc.subcore_barrier()
    pltpu.sync_copy(y_sh, y_full)

    nrows = ROWS_PER + jnp.where(sid == NS - 1, LANES, 0)

    @pl.when(cid == 0)
    def _add_base():
        def bb(j, carry):
            s = row0 + j * LANES
            acc_v[pl.ds(s, LANES)] = (
                acc_v[pl.ds(s, LANES)] + base_loc[pl.ds(j * LANES, LANES)])
            return carry

        lax.fori_loop(0, nrows // LANES, bb, 0)

    c_s.wait()
    c_d.wait()
    c_e.wait()

    @plsc.parallel_loop(0, E_PER // LANES, unroll=8)
    def _edges(i):
        off = i * LANES
        sv = src_v[pl.ds(off, LANES)]
        dv = dst_v[pl.ds(off, LANES)]
        ev = ea_v[pl.ds(off, LANES)]
        yv = plsc.load_gather(y_full, [sv])
        plsc.addupdate_scatter(acc_v, [dv], ev * yv)

    pltpu.sync_copy(acc_v, out_hbm.at[wid])


def kernel(x, edge_index, edge_attr, batch, W_rel, b_rel, W_root):
    xf = x.reshape(N_NODES * D_FEAT)
    ei = edge_index.astype(jnp.int32).reshape(2 * N_EDGES)
    wflat = jnp.concatenate(
        [W_rel[0], W_root[0], jnp.broadcast_to(b_rel[0], (LANES,))])

    partials = _sc_fused(xf, wflat, ei, edge_attr)

    out = pl.pallas_call(
        _reduce_body,
        out_shape=jax.ShapeDtypeStruct((N_NODES,), jnp.float32),
    )(partials)

    return out.reshape(N_NODES // CHUNK, CHUNK)
